# baseline (speedup 1.0000x reference)
import functools

import jax
import jax.numpy as jnp
from jax import lax
from jax.experimental import pallas as pl
from jax.experimental.pallas import tpu as pltpu
from jax.experimental.pallas import tpu_sc as plsc

_NUM_CLASSES = 19
_THING_LO = 11
_THING_HI = 18
_CENTER_THRESHOLD = 0.1
_NMS_PAD = 3
_TOP_K = 200
_H = 384
_W = 384
_P = _H * _W
_NEG_INF = float("-inf")
_SLAB = 8
_WIN = 40
_MAX_OFF = 14.0

_NC = 2
_NS = 16
_NW = _NC * _NS
_CHUNK = _P // _NW
_GROUPS = _CHUNK // 16
_SLICE = _P // _NS
_NROWS = 4 * _CHUNK // 128


def _sc_count_body(oy_hbm, ox_hbm, out_hbm, oyv, oxv, idxb, valb, zb, cnt_sh,
                   dma_sem):
    c = lax.axis_index("c")
    s = lax.axis_index("s")
    base = (c * _NS + s) * _CHUNK

    def zloop(i, carry):
        zb[pl.ds(i * 16, 16)] = jnp.zeros((16,), jnp.int32)
        return carry

    lax.fori_loop(0, _SLICE // 16, zloop, 0)
    pltpu.sync_copy(zb, cnt_sh.at[pl.ds(s * _SLICE, _SLICE)])
    plsc.subcore_barrier()

    pltpu.sync_copy(oy_hbm.at[pl.ds(base, _CHUNK)], oyv)
    pltpu.sync_copy(ox_hbm.at[pl.ds(base, _CHUNK)], oxv)

    lane = lax.iota(jnp.int32, 16)

    def do_group(g, half):
        pid0 = base + g * 16
        y0 = pid0 // _W
        x0 = pid0 - y0 * _W
        sy = y0.astype(jnp.float32) + oyv[pl.ds(g * 16, 16)]
        sx = (x0 + lane).astype(jnp.float32) + oxv[pl.ds(g * 16, 16)]
        fy = sy.astype(jnp.int32)
        fy = jnp.where(fy.astype(jnp.float32) > sy, fy - 1, fy)
        fx = sx.astype(jnp.int32)
        fx = jnp.where(fx.astype(jnp.float32) > sx, fx - 1, fx)
        r = g // 2
        c0 = half * 64
        for ii, (dy_, dx_) in enumerate(((0, 0), (0, 1), (1, 0), (1, 1))):
            iy = fy + dy_
            ix = fx + dx_
            inb = (iy >= 0) & (iy < _H) & (ix >= 0) & (ix < _W)
            dyf = sy - iy.astype(jnp.float32)
            dxf = sx - ix.astype(jnp.float32)
            ind = (dyf * dyf + dxf * dxf < 1.0) & inb
            val = jnp.where(ind, jnp.int32(1), jnp.int32(0))
            idx = (jnp.clip(iy, 0, _H - 1) * _W
                   + jnp.clip(ix, 0, _W - 1))
            idxb[r, pl.ds(c0 + ii * 16, 16)] = idx
            valb[r, pl.ds(c0 + ii * 16, 16)] = val

    def gloop(i, carry):
        do_group(i * 2, 0)
        do_group(i * 2 + 1, 1)
        pltpu.async_copy(valb.at[i], cnt_sh.at[idxb.at[i]], dma_sem,
                         add=True)
        return carry

    lax.fori_loop(0, _GROUPS // 2, gloop, 0)

    def dloop(j, carry):
        pltpu.make_async_copy(valb.at[j], cnt_sh.at[idxb.at[j]],
                              dma_sem).wait()
        return carry

    lax.fori_loop(0, _NROWS, dloop, 0)
    plsc.subcore_barrier()

    pltpu.sync_copy(cnt_sh.at[pl.ds(s * _SLICE, _SLICE)],
                    out_hbm.at[c, pl.ds(s * _SLICE, _SLICE)])


def _sc_count(oy, ox):
    run = functools.partial(
        pl.kernel,
        mesh=plsc.VectorSubcoreMesh(core_axis_name="c", subcore_axis_name="s",
                                    num_cores=_NC, num_subcores=_NS),
        out_type=jax.ShapeDtypeStruct((_NC, _P), jnp.int32),
        scratch_types=[
            pltpu.VMEM((_CHUNK,), jnp.float32),
            pltpu.VMEM((_CHUNK,), jnp.float32),
            pltpu.VMEM((_NROWS, 128), jnp.int32),
            pltpu.VMEM((_NROWS, 128), jnp.int32),
            pltpu.VMEM((_SLICE,), jnp.int32),
            pltpu.VMEM_SHARED((_P,), jnp.int32),
            pltpu.SemaphoreType.DMA,
        ],
    )(_sc_count_body)
    return run(oy, ox)


_ROWBLK = 48
_NBLK = _H // _ROWBLK


def _fuse_body(sem_hbm, heat_ref, off_ref, cntp_ref,
               pan_ref, semout_hbm, heatout_hbm, offout_hbm,
               sempred_ref, ascores_ref, sy_ref, sx_ref,
               semb0, semb1, dsem0, dsem1, psem):
    pltpu.make_async_copy(sem_hbm, semout_hbm, psem).start()
    pltpu.make_async_copy(heat_ref, heatout_hbm, psem).start()
    pltpu.make_async_copy(off_ref, offout_hbm, psem).start()

    bufs = (semb0, semb1)
    sems = (dsem0, dsem1)
    pltpu.make_async_copy(
        sem_hbm.at[:, pl.ds(0, _ROWBLK), :], semb0, dsem0).start()
    for blk in range(_NBLK):
        if blk + 1 < _NBLK:
            pltpu.make_async_copy(
                sem_hbm.at[:, pl.ds((blk + 1) * _ROWBLK, _ROWBLK), :],
                bufs[(blk + 1) % 2], sems[(blk + 1) % 2]).start()
        buf = bufs[blk % 2]
        pltpu.make_async_copy(
            sem_hbm.at[:, pl.ds(blk * _ROWBLK, _ROWBLK), :],
            buf, sems[blk % 2]).wait()
        x = buf[...]
        best = x[0]
        bidx = jnp.zeros(best.shape, jnp.int32)
        for c in range(1, _NUM_CLASSES):
            better = x[c] > best
            best = jnp.where(better, x[c], best)
            bidx = jnp.where(better, c, bidx)
        sempred_ref[pl.ds(blk * _ROWBLK, _ROWBLK), :] = bidx

    heat = heat_ref[...]

    ninf_rows = jnp.full((_NMS_PAD, _W), _NEG_INF, jnp.float32)
    hp = jnp.concatenate([ninf_rows, heat, ninf_rows], axis=0)
    rm = hp[0:_H, :]
    for d in range(1, 2 * _NMS_PAD + 1):
        rm = jnp.maximum(rm, hp[d:d + _H, :])
    ninf_cols = jnp.full((_H, _NMS_PAD), _NEG_INF, jnp.float32)
    cp = jnp.concatenate([ninf_cols, rm, ninf_cols], axis=1)
    nms = cp[:, 0:_W]
    for d in range(1, 2 * _NMS_PAD + 1):
        nms = jnp.maximum(nms, cp[:, d:d + _W])

    cmask = (heat > _CENTER_THRESHOLD) & (heat == nms)
    scores = jnp.where(cmask, heat, _NEG_INF)

    sempred = sempred_ref[...]
    cnt = cntp_ref[0] + cntp_ref[1]
    thing = (sempred >= _THING_LO) & (sempred <= _THING_HI)
    acc = cmask & (heat < jnp.inf) & (cnt >= 32) & thing
    nacc = jnp.sum(acc.astype(jnp.int32))

    pan_ref[...] = sempred

    @pl.when(nacc > 0)
    def _():
        ri = lax.broadcasted_iota(jnp.int32, (_H, _W), 0)
        ci = lax.broadcasted_iota(jnp.int32, (_H, _W), 1)
        fidx = ri * _W + ci
        sy_ref[...] = ri.astype(jnp.float32) + off_ref[0]
        sx_ref[...] = ci.astype(jnp.float32) + off_ref[1]
        ascores_ref[...] = jnp.where(acc, scores, _NEG_INF)
        has_outlier = jnp.max(jnp.abs(off_ref[0])) > _MAX_OFF
        big = jnp.int32(1 << 30)

        def cond(carry):
            return carry[0] == 1

        def body(carry):
            _, next_id = carry
            a = ascores_ref[...]
            m = jnp.max(a)
            idx = jnp.min(jnp.where(a == m, fidx, big))
            row = idx // _W
            col = idx % _W
            ascores_ref[...] = jnp.where(fidx == idx, _NEG_INF, a)
            scnt = (jnp.sum((scores > m).astype(jnp.int32))
                    + jnp.sum(((scores == m) & (fidx < idx))
                              .astype(jnp.int32)))
            ok = (m > _NEG_INF) & (scnt < _TOP_K)
            rowf = row.astype(jnp.float32)
            colf = col.astype(jnp.float32)

            @pl.when(ok & jnp.logical_not(has_outlier))
            def _():
                wbase = (row // _SLAB) * _SLAB - 2 * _SLAB
                wstart = pl.multiple_of(
                    jnp.clip(wbase, 0, _H - _WIN), _SLAB)
                dyw = sy_ref[pl.ds(wstart, _WIN), :] - rowf
                dxw = sx_ref[pl.ds(wstart, _WIN), :] - colf
                ind_w = dyw * dyw + dxw * dxw < 1.0
                panw = pan_ref[pl.ds(wstart, _WIN), :]
                pan_ref[pl.ds(wstart, _WIN), :] = (
                    jnp.where(ind_w, next_id, panw))

            @pl.when(ok & has_outlier)
            def _():
                dy = sy_ref[...] - rowf
                dx = sx_ref[...] - colf
                ind_f = dy * dy + dx * dx < 1.0
                pan_ref[...] = jnp.where(ind_f, next_id, pan_ref[...])

            return (ok.astype(jnp.int32), next_id + ok.astype(jnp.int32))

        lax.while_loop(cond, body, (jnp.int32(1), jnp.int32(1000)))

    pltpu.make_async_copy(sem_hbm, semout_hbm, psem).wait()
    pltpu.make_async_copy(heat_ref, heatout_hbm, psem).wait()
    pltpu.make_async_copy(off_ref, offout_hbm, psem).wait()


@jax.jit
def kernel(semantic_logits, center_heatmap, offset_map):
    sem = semantic_logits[0]
    heat = center_heatmap[0, 0]
    off = offset_map[0]

    partial_counts = _sc_count(off[0].reshape(_P), off[1].reshape(_P))
    cntp = partial_counts.reshape(_NC, _H, _W)

    pan, semc, heatc, offc = pl.pallas_call(
        _fuse_body,
        out_shape=[
            jax.ShapeDtypeStruct((_H, _W), jnp.int32),
            jax.ShapeDtypeStruct((_NUM_CLASSES, _H, _W), jnp.float32),
            jax.ShapeDtypeStruct((_H, _W), jnp.float32),
            jax.ShapeDtypeStruct((2, _H, _W), jnp.float32),
        ],
        in_specs=[
            pl.BlockSpec(memory_space=pl.ANY),
            pl.BlockSpec(memory_space=pltpu.VMEM),
            pl.BlockSpec(memory_space=pltpu.VMEM),
            pl.BlockSpec(memory_space=pltpu.VMEM),
        ],
        out_specs=[
            pl.BlockSpec(memory_space=pltpu.VMEM),
            pl.BlockSpec(memory_space=pl.ANY),
            pl.BlockSpec(memory_space=pl.ANY),
            pl.BlockSpec(memory_space=pl.ANY),
        ],
        scratch_shapes=[
            pltpu.VMEM((_H, _W), jnp.int32),
            pltpu.VMEM((_H, _W), jnp.float32),
            pltpu.VMEM((_H, _W), jnp.float32),
            pltpu.VMEM((_H, _W), jnp.float32),
            pltpu.VMEM((_NUM_CLASSES, _ROWBLK, _W), jnp.float32),
            pltpu.VMEM((_NUM_CLASSES, _ROWBLK, _W), jnp.float32),
            pltpu.SemaphoreType.DMA,
            pltpu.SemaphoreType.DMA,
            pltpu.SemaphoreType.DMA,
        ],
    )(sem, heat, off, cntp)

    return (semc[None], heatc[None, None], offc[None], pan[None])


# speedup vs baseline: 6.5644x; 6.5644x over previous
import functools

import jax
import jax.numpy as jnp
from jax import lax
from jax.experimental import pallas as pl
from jax.experimental.pallas import tpu as pltpu
from jax.experimental.pallas import tpu_sc as plsc

_NUM_CLASSES = 19
_THING_LO = 11
_THING_HI = 18
_CENTER_THRESHOLD = 0.1
_NMS_PAD = 3
_TOP_K = 200
_H = 384
_W = 384
_P = _H * _W
_NEG_INF = float("-inf")
_SLAB = 8
_WIN = 40
_MAX_OFF = 14.0

_NC = 2
_NS = 16
_NW = _NC * _NS
_CHUNK = _P // _NW
_GROUPS = _CHUNK // 16
_SLICE = _P // _NS
_NROWS = 4 * _CHUNK // 128


def _sc_count_body(oy_hbm, ox_hbm, out_hbm, oyv, oxv, idxb, valb, zb, cnt_sh,
                   dma_sem):
    c = lax.axis_index("c")
    s = lax.axis_index("s")
    base = (c * _NS + s) * _CHUNK

    def zloop(i, carry):
        zb[pl.ds(i * 16, 16)] = jnp.zeros((16,), jnp.int32)
        return carry

    lax.fori_loop(0, _SLICE // 16, zloop, 0)
    pltpu.sync_copy(zb, cnt_sh.at[pl.ds(s * _SLICE, _SLICE)])
    plsc.subcore_barrier()

    pltpu.sync_copy(oy_hbm.at[pl.ds(base, _CHUNK)], oyv)
    pltpu.sync_copy(ox_hbm.at[pl.ds(base, _CHUNK)], oxv)

    lane = lax.iota(jnp.int32, 16)

    def do_group(g, half):
        pid0 = base + g * 16
        y0 = pid0 // _W
        x0 = pid0 - y0 * _W
        sy = y0.astype(jnp.float32) + oyv[pl.ds(g * 16, 16)]
        sx = (x0 + lane).astype(jnp.float32) + oxv[pl.ds(g * 16, 16)]
        fy = sy.astype(jnp.int32)
        fy = jnp.where(fy.astype(jnp.float32) > sy, fy - 1, fy)
        fx = sx.astype(jnp.int32)
        fx = jnp.where(fx.astype(jnp.float32) > sx, fx - 1, fx)
        r = g // 2
        c0 = half * 64
        for ii, (dy_, dx_) in enumerate(((0, 0), (0, 1), (1, 0), (1, 1))):
            iy = fy + dy_
            ix = fx + dx_
            inb = (iy >= 0) & (iy < _H) & (ix >= 0) & (ix < _W)
            dyf = sy - iy.astype(jnp.float32)
            dxf = sx - ix.astype(jnp.float32)
            ind = (dyf * dyf + dxf * dxf < 1.0) & inb
            val = jnp.where(ind, jnp.int32(1), jnp.int32(0))
            idx = (jnp.clip(iy, 0, _H - 1) * _W
                   + jnp.clip(ix, 0, _W - 1))
            idxb[r, pl.ds(c0 + ii * 16, 16)] = idx
            valb[r, pl.ds(c0 + ii * 16, 16)] = val

    def gloop(i, carry):
        do_group(i * 2, 0)
        do_group(i * 2 + 1, 1)
        pltpu.async_copy(valb.at[i], cnt_sh.at[idxb.at[i]], dma_sem,
                         add=True)
        return carry

    lax.fori_loop(0, _GROUPS // 2, gloop, 0)

    def dloop(j, carry):
        pltpu.make_async_copy(valb.at[j], cnt_sh.at[idxb.at[j]],
                              dma_sem).wait()
        return carry

    lax.fori_loop(0, _NROWS, dloop, 0)
    plsc.subcore_barrier()

    pltpu.sync_copy(cnt_sh.at[pl.ds(s * _SLICE, _SLICE)],
                    out_hbm.at[c, pl.ds(s * _SLICE, _SLICE)])


def _sc_count(oy, ox):
    run = functools.partial(
        pl.kernel,
        mesh=plsc.VectorSubcoreMesh(core_axis_name="c", subcore_axis_name="s",
                                    num_cores=_NC, num_subcores=_NS),
        out_type=jax.ShapeDtypeStruct((_NC, _P), jnp.int32),
        scratch_types=[
            pltpu.VMEM((_CHUNK,), jnp.float32),
            pltpu.VMEM((_CHUNK,), jnp.float32),
            pltpu.VMEM((_NROWS, 128), jnp.int32),
            pltpu.VMEM((_NROWS, 128), jnp.int32),
            pltpu.VMEM((_SLICE,), jnp.int32),
            pltpu.VMEM_SHARED((_P,), jnp.int32),
            pltpu.SemaphoreType.DMA,
        ],
    )(_sc_count_body)
    return run(oy, ox)


_ROWBLK = 48
_NBLK = _H // _ROWBLK


def _fuse_body(sem_hbm, heat_ref, off_ref, cntp_ref, pan_ref,
               sempred_ref, ascores_ref, sy_ref, sx_ref,
               semb0, semb1, dsem0, dsem1):
    bufs = (semb0, semb1)
    sems = (dsem0, dsem1)
    pltpu.make_async_copy(
        sem_hbm.at[:, pl.ds(0, _ROWBLK), :], semb0, dsem0).start()
    for blk in range(_NBLK):
        if blk + 1 < _NBLK:
            pltpu.make_async_copy(
                sem_hbm.at[:, pl.ds((blk + 1) * _ROWBLK, _ROWBLK), :],
                bufs[(blk + 1) % 2], sems[(blk + 1) % 2]).start()
        buf = bufs[blk % 2]
        pltpu.make_async_copy(
            sem_hbm.at[:, pl.ds(blk * _ROWBLK, _ROWBLK), :],
            buf, sems[blk % 2]).wait()
        x = buf[...]
        best = x[0]
        bidx = jnp.zeros(best.shape, jnp.int32)
        for c in range(1, _NUM_CLASSES):
            better = x[c] > best
            best = jnp.where(better, x[c], best)
            bidx = jnp.where(better, c, bidx)
        sempred_ref[pl.ds(blk * _ROWBLK, _ROWBLK), :] = bidx

    heat = heat_ref[...]

    ninf_rows = jnp.full((_NMS_PAD, _W), _NEG_INF, jnp.float32)
    hp = jnp.concatenate([ninf_rows, heat, ninf_rows], axis=0)
    rm = hp[0:_H, :]
    for d in range(1, 2 * _NMS_PAD + 1):
        rm = jnp.maximum(rm, hp[d:d + _H, :])
    ninf_cols = jnp.full((_H, _NMS_PAD), _NEG_INF, jnp.float32)
    cp = jnp.concatenate([ninf_cols, rm, ninf_cols], axis=1)
    nms = cp[:, 0:_W]
    for d in range(1, 2 * _NMS_PAD + 1):
        nms = jnp.maximum(nms, cp[:, d:d + _W])

    cmask = (heat > _CENTER_THRESHOLD) & (heat == nms)
    scores = jnp.where(cmask, heat, _NEG_INF)

    sempred = sempred_ref[...]
    cnt = cntp_ref[0] + cntp_ref[1]
    thing = (sempred >= _THING_LO) & (sempred <= _THING_HI)
    acc = cmask & (heat < jnp.inf) & (cnt >= 32) & thing
    nacc = jnp.sum(acc.astype(jnp.int32))

    pan_ref[...] = sempred

    @pl.when(nacc > 0)
    def _():
        ri = lax.broadcasted_iota(jnp.int32, (_H, _W), 0)
        ci = lax.broadcasted_iota(jnp.int32, (_H, _W), 1)
        fidx = ri * _W + ci
        sy_ref[...] = ri.astype(jnp.float32) + off_ref[0]
        sx_ref[...] = ci.astype(jnp.float32) + off_ref[1]
        ascores_ref[...] = jnp.where(acc, scores, _NEG_INF)
        has_outlier = jnp.max(jnp.abs(off_ref[0])) > _MAX_OFF
        big = jnp.int32(1 << 30)

        def cond(carry):
            return carry[0] == 1

        def body(carry):
            _, next_id = carry
            a = ascores_ref[...]
            m = jnp.max(a)
            idx = jnp.min(jnp.where(a == m, fidx, big))
            row = idx // _W
            col = idx % _W
            ascores_ref[...] = jnp.where(fidx == idx, _NEG_INF, a)
            scnt = (jnp.sum((scores > m).astype(jnp.int32))
                    + jnp.sum(((scores == m) & (fidx < idx))
                              .astype(jnp.int32)))
            ok = (m > _NEG_INF) & (scnt < _TOP_K)
            rowf = row.astype(jnp.float32)
            colf = col.astype(jnp.float32)

            @pl.when(ok & jnp.logical_not(has_outlier))
            def _():
                wbase = (row // _SLAB) * _SLAB - 2 * _SLAB
                wstart = pl.multiple_of(
                    jnp.clip(wbase, 0, _H - _WIN), _SLAB)
                dyw = sy_ref[pl.ds(wstart, _WIN), :] - rowf
                dxw = sx_ref[pl.ds(wstart, _WIN), :] - colf
                ind_w = dyw * dyw + dxw * dxw < 1.0
                panw = pan_ref[pl.ds(wstart, _WIN), :]
                pan_ref[pl.ds(wstart, _WIN), :] = (
                    jnp.where(ind_w, next_id, panw))

            @pl.when(ok & has_outlier)
            def _():
                dy = sy_ref[...] - rowf
                dx = sx_ref[...] - colf
                ind_f = dy * dy + dx * dx < 1.0
                pan_ref[...] = jnp.where(ind_f, next_id, pan_ref[...])

            return (ok.astype(jnp.int32), next_id + ok.astype(jnp.int32))

        lax.while_loop(cond, body, (jnp.int32(1), jnp.int32(1000)))


@jax.jit
def kernel(semantic_logits, center_heatmap, offset_map):
    sem = semantic_logits[0]
    heat = center_heatmap[0, 0]
    off = offset_map[0]

    partial_counts = _sc_count(off[0].reshape(_P), off[1].reshape(_P))
    cntp = partial_counts.reshape(_NC, _H, _W)

    pan = pl.pallas_call(
        _fuse_body,
        out_shape=jax.ShapeDtypeStruct((_H, _W), jnp.int32),
        in_specs=[
            pl.BlockSpec(memory_space=pl.ANY),
            pl.BlockSpec(memory_space=pltpu.VMEM),
            pl.BlockSpec(memory_space=pltpu.VMEM),
            pl.BlockSpec(memory_space=pltpu.VMEM),
        ],
        scratch_shapes=[
            pltpu.VMEM((_H, _W), jnp.int32),
            pltpu.VMEM((_H, _W), jnp.float32),
            pltpu.VMEM((_H, _W), jnp.float32),
            pltpu.VMEM((_H, _W), jnp.float32),
            pltpu.VMEM((_NUM_CLASSES, _ROWBLK, _W), jnp.float32),
            pltpu.VMEM((_NUM_CLASSES, _ROWBLK, _W), jnp.float32),
            pltpu.SemaphoreType.DMA,
            pltpu.SemaphoreType.DMA,
        ],
    )(sem, heat, off, cntp)

    return (semantic_logits, center_heatmap, offset_map, pan[None])


# speedup vs baseline: 7.1923x; 1.0957x over previous
import functools

import jax
import jax.numpy as jnp
from jax import lax
from jax.experimental import pallas as pl
from jax.experimental.pallas import tpu as pltpu
from jax.experimental.pallas import tpu_sc as plsc

_NUM_CLASSES = 19
_THING_LO = 11
_THING_HI = 18
_CENTER_THRESHOLD = 0.1
_NMS_PAD = 3
_TOP_K = 200
_H = 384
_W = 384
_P = _H * _W
_NEG_INF = float("-inf")
_SLAB = 8
_WIN = 40
_MAX_OFF = 14.0

_NC = 2
_NS = 16
_NW = _NC * _NS
_CHUNK = _P // _NW
_GROUPS = _CHUNK // 16
_SLICE = _P // _NS
_NROWS = 4 * _CHUNK // 128


def _sc_count_body(oy_hbm, ox_hbm, out_hbm, oyv, oxv, idxb, valb, zb, cnt_sh,
                   dma_sem):
    c = lax.axis_index("c")
    s = lax.axis_index("s")
    base = (c * _NS + s) * _CHUNK

    def zloop(i, carry):
        zb[pl.ds(i * 16, 16)] = jnp.zeros((16,), jnp.int32)
        return carry

    lax.fori_loop(0, _SLICE // 16, zloop, 0)
    pltpu.sync_copy(zb, cnt_sh.at[pl.ds(s * _SLICE, _SLICE)])
    plsc.subcore_barrier()

    pltpu.sync_copy(oy_hbm.at[pl.ds(base, _CHUNK)], oyv)
    pltpu.sync_copy(ox_hbm.at[pl.ds(base, _CHUNK)], oxv)

    lane = lax.iota(jnp.int32, 16)

    def do_group(g, half):
        pid0 = base + g * 16
        y0 = pid0 // _W
        x0 = pid0 - y0 * _W
        sy = y0.astype(jnp.float32) + oyv[pl.ds(g * 16, 16)]
        sx = (x0 + lane).astype(jnp.float32) + oxv[pl.ds(g * 16, 16)]
        fy = sy.astype(jnp.int32)
        fy = jnp.where(fy.astype(jnp.float32) > sy, fy - 1, fy)
        fx = sx.astype(jnp.int32)
        fx = jnp.where(fx.astype(jnp.float32) > sx, fx - 1, fx)
        r = g // 2
        c0 = half * 64
        for ii, (dy_, dx_) in enumerate(((0, 0), (0, 1), (1, 0), (1, 1))):
            iy = fy + dy_
            ix = fx + dx_
            inb = (iy >= 0) & (iy < _H) & (ix >= 0) & (ix < _W)
            dyf = sy - iy.astype(jnp.float32)
            dxf = sx - ix.astype(jnp.float32)
            ind = (dyf * dyf + dxf * dxf < 1.0) & inb
            val = jnp.where(ind, jnp.int32(1), jnp.int32(0))
            idx = (jnp.clip(iy, 0, _H - 1) * _W
                   + jnp.clip(ix, 0, _W - 1))
            idxb[r, pl.ds(c0 + ii * 16, 16)] = idx
            valb[r, pl.ds(c0 + ii * 16, 16)] = val

    def gloop(i, carry):
        do_group(i * 2, 0)
        do_group(i * 2 + 1, 1)
        pltpu.async_copy(valb.at[i], cnt_sh.at[idxb.at[i]], dma_sem,
                         add=True)
        return carry

    lax.fori_loop(0, _GROUPS // 2, gloop, 0)

    def dloop(j, carry):
        pltpu.make_async_copy(valb.at[j], cnt_sh.at[idxb.at[j]],
                              dma_sem).wait()
        return carry

    lax.fori_loop(0, _NROWS, dloop, 0)
    plsc.subcore_barrier()

    pltpu.sync_copy(cnt_sh.at[pl.ds(s * _SLICE, _SLICE)],
                    out_hbm.at[c, pl.ds(s * _SLICE, _SLICE)])


def _sc_count(oy, ox):
    run = functools.partial(
        pl.kernel,
        mesh=plsc.VectorSubcoreMesh(core_axis_name="c", subcore_axis_name="s",
                                    num_cores=_NC, num_subcores=_NS),
        out_type=jax.ShapeDtypeStruct((_NC, _P), jnp.int32),
        scratch_types=[
            pltpu.VMEM((_CHUNK,), jnp.float32),
            pltpu.VMEM((_CHUNK,), jnp.float32),
            pltpu.VMEM((_NROWS, 128), jnp.int32),
            pltpu.VMEM((_NROWS, 128), jnp.int32),
            pltpu.VMEM((_SLICE,), jnp.int32),
            pltpu.VMEM_SHARED((_P,), jnp.int32),
            pltpu.SemaphoreType.DMA,
        ],
    )(_sc_count_body)
    return run(oy, ox)


_ROWBLK = 48
_NBLK = _H // _ROWBLK


def _fuse_body(sem_hbm, heat_ref, off_ref, cntp_ref,
               pan_ref, semout_hbm, heatout_hbm, offout_hbm,
               sempred_ref, ascores_ref, sy_ref, sx_ref,
               semb0, semb1, dsem0, dsem1, wsem0, wsem1, psem):
    pltpu.make_async_copy(heat_ref, heatout_hbm, psem).start()
    pltpu.make_async_copy(off_ref, offout_hbm, psem).start()

    bufs = (semb0, semb1)
    sems = (dsem0, dsem1)
    wsems = (wsem0, wsem1)

    def _chunk(i):
        return (slice(None), pl.ds(i * _ROWBLK, _ROWBLK), slice(None))

    pltpu.make_async_copy(sem_hbm.at[_chunk(0)], semb0, dsem0).start()
    for blk in range(_NBLK):
        if blk + 1 < _NBLK:
            nbuf = bufs[(blk + 1) % 2]
            if blk >= 1:
                pltpu.make_async_copy(
                    nbuf, semout_hbm.at[_chunk(blk - 1)],
                    wsems[(blk + 1) % 2]).wait()
            pltpu.make_async_copy(
                sem_hbm.at[_chunk(blk + 1)], nbuf, sems[(blk + 1) % 2]
            ).start()
        buf = bufs[blk % 2]
        pltpu.make_async_copy(
            sem_hbm.at[_chunk(blk)], buf, sems[blk % 2]).wait()
        x = buf[...]
        best = x[0]
        bidx = jnp.zeros(best.shape, jnp.int32)
        for c in range(1, _NUM_CLASSES):
            better = x[c] > best
            best = jnp.where(better, x[c], best)
            bidx = jnp.where(better, c, bidx)
        sempred_ref[pl.ds(blk * _ROWBLK, _ROWBLK), :] = bidx
        pltpu.make_async_copy(
            buf, semout_hbm.at[_chunk(blk)], wsems[blk % 2]).start()

    heat = heat_ref[...]

    ninf_rows = jnp.full((_NMS_PAD, _W), _NEG_INF, jnp.float32)
    hp = jnp.concatenate([ninf_rows, heat, ninf_rows], axis=0)
    rm = hp[0:_H, :]
    for d in range(1, 2 * _NMS_PAD + 1):
        rm = jnp.maximum(rm, hp[d:d + _H, :])
    ninf_cols = jnp.full((_H, _NMS_PAD), _NEG_INF, jnp.float32)
    cp = jnp.concatenate([ninf_cols, rm, ninf_cols], axis=1)
    nms = cp[:, 0:_W]
    for d in range(1, 2 * _NMS_PAD + 1):
        nms = jnp.maximum(nms, cp[:, d:d + _W])

    cmask = (heat > _CENTER_THRESHOLD) & (heat == nms)
    scores = jnp.where(cmask, heat, _NEG_INF)

    sempred = sempred_ref[...]
    cnt = cntp_ref[0] + cntp_ref[1]
    thing = (sempred >= _THING_LO) & (sempred <= _THING_HI)
    acc = cmask & (heat < jnp.inf) & (cnt >= 32) & thing
    nacc = jnp.sum(acc.astype(jnp.int32))

    pan_ref[...] = sempred

    @pl.when(nacc > 0)
    def _():
        ri = lax.broadcasted_iota(jnp.int32, (_H, _W), 0)
        ci = lax.broadcasted_iota(jnp.int32, (_H, _W), 1)
        fidx = ri * _W + ci
        sy_ref[...] = ri.astype(jnp.float32) + off_ref[0]
        sx_ref[...] = ci.astype(jnp.float32) + off_ref[1]
        ascores_ref[...] = jnp.where(acc, scores, _NEG_INF)
        has_outlier = jnp.max(jnp.abs(off_ref[0])) > _MAX_OFF
        big = jnp.int32(1 << 30)

        def cond(carry):
            return carry[0] == 1

        def body(carry):
            _, next_id = carry
            a = ascores_ref[...]
            m = jnp.max(a)
            idx = jnp.min(jnp.where(a == m, fidx, big))
            row = idx // _W
            col = idx % _W
            ascores_ref[...] = jnp.where(fidx == idx, _NEG_INF, a)
            scnt = (jnp.sum((scores > m).astype(jnp.int32))
                    + jnp.sum(((scores == m) & (fidx < idx))
                              .astype(jnp.int32)))
            ok = (m > _NEG_INF) & (scnt < _TOP_K)
            rowf = row.astype(jnp.float32)
            colf = col.astype(jnp.float32)

            @pl.when(ok & jnp.logical_not(has_outlier))
            def _():
                wbase = (row // _SLAB) * _SLAB - 2 * _SLAB
                wstart = pl.multiple_of(
                    jnp.clip(wbase, 0, _H - _WIN), _SLAB)
                dyw = sy_ref[pl.ds(wstart, _WIN), :] - rowf
                dxw = sx_ref[pl.ds(wstart, _WIN), :] - colf
                ind_w = dyw * dyw + dxw * dxw < 1.0
                panw = pan_ref[pl.ds(wstart, _WIN), :]
                pan_ref[pl.ds(wstart, _WIN), :] = (
                    jnp.where(ind_w, next_id, panw))

            @pl.when(ok & has_outlier)
            def _():
                dy = sy_ref[...] - rowf
                dx = sx_ref[...] - colf
                ind_f = dy * dy + dx * dx < 1.0
                pan_ref[...] = jnp.where(ind_f, next_id, pan_ref[...])

            return (ok.astype(jnp.int32), next_id + ok.astype(jnp.int32))

        lax.while_loop(cond, body, (jnp.int32(1), jnp.int32(1000)))

    pltpu.make_async_copy(semb0, semout_hbm.at[:, pl.ds(6 * _ROWBLK, _ROWBLK), :],
                          wsem0).wait()
    pltpu.make_async_copy(semb1, semout_hbm.at[:, pl.ds(7 * _ROWBLK, _ROWBLK), :],
                          wsem1).wait()
    pltpu.make_async_copy(heat_ref, heatout_hbm, psem).wait()
    pltpu.make_async_copy(off_ref, offout_hbm, psem).wait()


@jax.jit
def kernel(semantic_logits, center_heatmap, offset_map):
    sem = semantic_logits[0]
    heat = center_heatmap[0, 0]
    off = offset_map[0]

    partial_counts = _sc_count(off[0].reshape(_P), off[1].reshape(_P))
    cntp = partial_counts.reshape(_NC, _H, _W)

    pan, semc, heatc, offc = pl.pallas_call(
        _fuse_body,
        out_shape=[
            jax.ShapeDtypeStruct((_H, _W), jnp.int32),
            jax.ShapeDtypeStruct((_NUM_CLASSES, _H, _W), jnp.float32),
            jax.ShapeDtypeStruct((_H, _W), jnp.float32),
            jax.ShapeDtypeStruct((2, _H, _W), jnp.float32),
        ],
        in_specs=[
            pl.BlockSpec(memory_space=pl.ANY),
            pl.BlockSpec(memory_space=pltpu.VMEM),
            pl.BlockSpec(memory_space=pltpu.VMEM),
            pl.BlockSpec(memory_space=pltpu.VMEM),
        ],
        out_specs=[
            pl.BlockSpec(memory_space=pltpu.VMEM),
            pl.BlockSpec(memory_space=pl.ANY),
            pl.BlockSpec(memory_space=pl.ANY),
            pl.BlockSpec(memory_space=pl.ANY),
        ],
        scratch_shapes=[
            pltpu.VMEM((_H, _W), jnp.int32),
            pltpu.VMEM((_H, _W), jnp.float32),
            pltpu.VMEM((_H, _W), jnp.float32),
            pltpu.VMEM((_H, _W), jnp.float32),
            pltpu.VMEM((_NUM_CLASSES, _ROWBLK, _W), jnp.float32),
            pltpu.VMEM((_NUM_CLASSES, _ROWBLK, _W), jnp.float32),
            pltpu.SemaphoreType.DMA,
            pltpu.SemaphoreType.DMA,
            pltpu.SemaphoreType.DMA,
            pltpu.SemaphoreType.DMA,
            pltpu.SemaphoreType.DMA,
        ],
    )(sem, heat, off, cntp)

    return (semc[None], heatc[None, None], offc[None], pan[None])
